# Initial kernel scaffold; baseline (speedup 1.0000x reference)
import jax
import jax.numpy as jnp
from jax.experimental import pallas as pl


def kernel(x, duration, max_len):
    raise NotImplementedError("write your pallas kernel here")


# speedup vs baseline: 78.3186x; 78.3186x over previous
import jax
import jax.numpy as jnp
from jax.experimental import pallas as pl


def _scan_kernel(ml_ref, dur_ref, starts_ref, ends_ref):
    T = dur_ref.shape[0]
    ml = ml_ref[...]

    def body(t, pos):
        d = dur_ref[pl.ds(t, 1), :]
        fits = (d > 0) & ((pos + d) <= ml)
        nd = pos + jnp.where(fits, d, 0)
        starts_ref[pl.ds(t, 1), :] = pos
        ends_ref[pl.ds(t, 1), :] = nd
        return nd

    pos0 = jnp.zeros_like(ml)
    jax.lax.fori_loop(0, T, body, pos0)


def _expand_kernel(x_ref, s_ref, e_ref, out_ref):
    T = x_ref.shape[2]
    L = out_ref.shape[2]
    s = s_ref[0]
    e = e_ref[0]
    p = jax.lax.broadcasted_iota(jnp.int32, (T, L), 1)
    g = ((p >= s) & (p < e)).astype(x_ref.dtype)
    out_ref[0] = jax.lax.dot_general(
        x_ref[0], g, (((1,), (0,)), ((), ())),
        preferred_element_type=jnp.float32)


def kernel(x, duration, max_len):
    B, C, T = x.shape
    try:
        L = int(max_len)
    except (TypeError, jax.errors.TracerIntegerConversionError):
        L = 2048

    dur_tb = duration.astype(jnp.int32).T
    ml = jnp.broadcast_to(jnp.asarray(max_len, jnp.int32), (1, B))

    starts_tb, ends_tb = pl.pallas_call(
        _scan_kernel,
        out_shape=[jax.ShapeDtypeStruct((T, B), jnp.int32)] * 2,
    )(ml, dur_tb)

    s = starts_tb.T.reshape(B, T, 1)
    e = ends_tb.T.reshape(B, T, 1)

    out = pl.pallas_call(
        _expand_kernel,
        grid=(B,),
        in_specs=[
            pl.BlockSpec((1, C, T), lambda b: (b, 0, 0)),
            pl.BlockSpec((1, T, 1), lambda b: (b, 0, 0)),
            pl.BlockSpec((1, T, 1), lambda b: (b, 0, 0)),
        ],
        out_specs=pl.BlockSpec((1, C, L), lambda b: (b, 0, 0)),
        out_shape=jax.ShapeDtypeStruct((B, C, L), x.dtype),
    )(x, s, e)
    return out
